# Initial kernel scaffold; baseline (speedup 1.0000x reference)
import jax
import jax.numpy as jnp
from jax.experimental import pallas as pl


def kernel(raw_features, neigh, weights, nodes_batch):
    raise NotImplementedError("write your pallas kernel here")


# speedup vs baseline: 2.0479x; 2.0479x over previous
import functools

import jax
import jax.numpy as jnp
from jax import lax
from jax.experimental import pallas as pl
from jax.experimental.pallas import tpu as pltpu
from jax.experimental.pallas import tpu_sc as plsc

N_NODES = 10000
DEG = 32
D_FEAT = 128
NUM_LAYERS = 2

_NC = 2
_NS = 16
_NW = _NC * _NS

_G = 4
_E = _G * DEG
_CHUNKS = (N_NODES + _G - 1) // _G
_CPW = (_CHUNKS + _NW - 1) // _NW
_LANES = 16
_NSLICE = D_FEAT // _LANES


def _layer_kernel(h_hbm, neigh_hbm, w_hbm, out_hbm, idx_v, w_v, rows_v, out_v,
                  sem):
    wid = lax.axis_index("s") * _NC + lax.axis_index("c")

    def chunk_body(t, carry):
        chunk = wid + t * _NW

        @pl.when(chunk < _CHUNKS)
        def _():
            base = chunk * _G
            pltpu.sync_copy(neigh_hbm.at[pl.ds(base * DEG, _E)], idx_v)
            pltpu.sync_copy(w_hbm.at[pl.ds(base * DEG, _E)], w_v)
            pltpu.async_copy(h_hbm.at[idx_v], rows_v, sem).wait()
            for g in range(_G):
                accs = [jnp.zeros((_LANES,), jnp.float32)
                        for _ in range(_NSLICE)]
                for jg in range(DEG // _LANES):
                    wv = w_v[pl.ds(g * DEG + jg * _LANES, _LANES)]
                    for j in range(_LANES):
                        e = g * DEG + jg * _LANES + j
                        w = wv[j]
                        for k in range(_NSLICE):
                            accs[k] = accs[k] + (
                                rows_v[e, pl.ds(k * _LANES, _LANES)] * w)
                for k in range(_NSLICE):
                    out_v[g, pl.ds(k * _LANES, _LANES)] = accs[k]
            pltpu.sync_copy(out_v, out_hbm.at[pl.ds(base, _G)])

        return carry

    lax.fori_loop(0, _CPW, chunk_body, 0)


@jax.jit
def _run(raw_features, neigh_flat, w_flat):
    mesh = plsc.VectorSubcoreMesh(core_axis_name="c", subcore_axis_name="s")
    layer = pl.kernel(
        _layer_kernel,
        mesh=mesh,
        out_type=jax.ShapeDtypeStruct((N_NODES, D_FEAT), jnp.float32),
        scratch_types=[
            pltpu.VMEM((_E,), jnp.int32),
            pltpu.VMEM((_E,), jnp.float32),
            pltpu.VMEM((_E, D_FEAT), jnp.float32),
            pltpu.VMEM((_G, D_FEAT), jnp.float32),
            pltpu.SemaphoreType.DMA,
        ],
    )
    h = raw_features
    for _ in range(NUM_LAYERS):
        h = layer(h, neigh_flat, w_flat)
    return h


def kernel(raw_features, neigh, weights, nodes_batch):
    del nodes_batch
    neigh_flat = neigh.reshape(-1).astype(jnp.int32)
    w_flat = weights.reshape(-1).astype(jnp.float32)
    return _run(raw_features, neigh_flat, w_flat)
